# baseline (speedup 1.0000x reference)
import jax
import jax.numpy as jnp
from jax.experimental import pallas as pl
from jax.experimental.pallas import tpu as pltpu
from jax.experimental.pallas import tpu_sc as plsc

N = 8192
D = 128
NS = 16

_LIN_BLK = 512
_KNN_BLK = 512
_GATHER_WIN = 128


def _linear_kernel(x_ref, w_ref, b_ref, o_ref):
    xb = x_ref[...].astype(jnp.bfloat16)
    wb = w_ref[...].astype(jnp.bfloat16)
    acc = jax.lax.dot_general(xb, wb, (((1,), (1,)), ((), ())),
                              preferred_element_type=jnp.float32)
    o_ref[...] = acc + b_ref[...]


def _linear(x, w, b):
    nblk = x.shape[0] // _LIN_BLK
    return pl.pallas_call(
        _linear_kernel,
        grid=(nblk,),
        in_specs=[
            pl.BlockSpec((_LIN_BLK, D), lambda i: (i, 0)),
            pl.BlockSpec((D, D), lambda i: (0, 0)),
            pl.BlockSpec((1, D), lambda i: (0, 0)),
        ],
        out_specs=pl.BlockSpec((_LIN_BLK, D), lambda i: (i, 0)),
        out_shape=jax.ShapeDtypeStruct((x.shape[0], D), jnp.float32),
        compiler_params=pltpu.CompilerParams(
            dimension_semantics=("parallel",)),
    )(x, w, b)


def _knn_kernel(p_ref, pbt_ref, pt_ref, idx_ref):
    B = p_ref.shape[0]
    p_blk = p_ref[...]
    sq_q = jnp.sum(p_blk * p_blk, axis=1, keepdims=True)
    pt = pt_ref[...]
    sq_k = jnp.sum(pt * pt, axis=0, keepdims=True)

    qb = p_blk.astype(jnp.bfloat16).astype(jnp.float32)
    kb = pbt_ref[...].astype(jnp.float32)
    g = qb[:, 0:1] * kb[0:1, :]
    g = g + qb[:, 1:2] * kb[1:2, :]
    g = g + qb[:, 2:3] * kb[2:3, :]
    d2 = (sq_q + sq_k) - 2.0 * g

    iota_f = jax.lax.broadcasted_iota(jnp.int32, (B, N), 1).astype(jnp.float32)
    big_f = jnp.float32(1e9)
    cols = []
    for t in range(NS):
        m = jnp.min(d2, axis=1, keepdims=True)
        cand = jnp.where(d2 <= m, iota_f, big_f)
        mi = jnp.min(cand, axis=1, keepdims=True)
        cols.append(mi)
        if t < NS - 1:
            d2 = jnp.where(cand == mi, jnp.float32(jnp.inf), d2)
    idx_ref[...] = jnp.concatenate(cols, axis=1).astype(jnp.int32)


def _knn(p, pbt, pt):
    nblk = N // _KNN_BLK
    return pl.pallas_call(
        _knn_kernel,
        grid=(nblk,),
        in_specs=[
            pl.BlockSpec((_KNN_BLK, 3), lambda i: (i, 0)),
            pl.BlockSpec((3, N), lambda i: (0, 0)),
            pl.BlockSpec((3, N), lambda i: (0, 0)),
        ],
        out_specs=pl.BlockSpec((_KNN_BLK, NS), lambda i: (i, 0)),
        out_shape=jax.ShapeDtypeStruct((N, NS), jnp.int32),
        compiler_params=pltpu.CompilerParams(
            dimension_semantics=("parallel",)),
    )(p, pbt, pt)


def _sc_gather(table, idx_flat):
    num_indices = idx_flat.shape[1]
    mesh = plsc.VectorSubcoreMesh(core_axis_name="core",
                                  subcore_axis_name="subcore")

    @pl.kernel(
        out_type=jax.ShapeDtypeStruct((num_indices, D), table.dtype),
        mesh=mesh,
    )
    def _gather_kernel(x_hbm, i_hbm, o_hbm):
        def body(i_vmem, o_vmem):
            pltpu.sync_copy(x_hbm.at[i_vmem.at[0]], o_vmem)

        pltpu.emit_pipeline(
            body,
            grid=(num_indices // _GATHER_WIN,),
            in_specs=[pl.BlockSpec((1, _GATHER_WIN), index_map=lambda i: (0, i))],
            out_specs=[pl.BlockSpec((_GATHER_WIN, D), index_map=lambda i: (i, 0))],
            core_axis_name=("core", "subcore"),
            dimension_semantics=(pltpu.PARALLEL,),
        )(i_hbm, o_hbm)

    return _gather_kernel(table, idx_flat)


def _lap_varphi_kernel(un_ref, up_ref, w_ref, b_ref, h_ref, stats_ref):
    un = un_ref[...]
    uq = up_ref[...]
    lap = jnp.mean(jax.nn.relu(un - uq[:, None, :]), axis=1)
    lb = lap.astype(jnp.bfloat16)
    wb = w_ref[...].astype(jnp.bfloat16)
    h = jax.lax.dot_general(lb, wb, (((1,), (1,)), ((), ())),
                            preferred_element_type=jnp.float32)
    h = h + b_ref[...]
    h_ref[...] = h
    stats_ref[0, 0, :] = jnp.sum(h, axis=0)
    stats_ref[0, 1, :] = jnp.sum(h * h, axis=0)


def _lap_varphi(u_n, u_pre, w_var, b_var):
    nblk = N // _LIN_BLK
    return pl.pallas_call(
        _lap_varphi_kernel,
        grid=(nblk,),
        in_specs=[
            pl.BlockSpec((_LIN_BLK, NS, D), lambda i: (i, 0, 0)),
            pl.BlockSpec((_LIN_BLK, D), lambda i: (i, 0)),
            pl.BlockSpec((D, D), lambda i: (0, 0)),
            pl.BlockSpec((1, D), lambda i: (0, 0)),
        ],
        out_specs=[
            pl.BlockSpec((_LIN_BLK, D), lambda i: (i, 0)),
            pl.BlockSpec((1, 2, D), lambda i: (i, 0, 0)),
        ],
        out_shape=[
            jax.ShapeDtypeStruct((N, D), jnp.float32),
            jax.ShapeDtypeStruct((nblk, 2, D), jnp.float32),
        ],
        compiler_params=pltpu.CompilerParams(
            dimension_semantics=("parallel",)),
    )(u_n, u_pre, w_var, b_var)


def _bn_kernel(h_ref, stats_ref, g_ref, b_ref, o_ref):
    stats = stats_ref[...]
    mean = jnp.sum(stats[:, 0, :], axis=0, keepdims=True) / N
    ex2 = jnp.sum(stats[:, 1, :], axis=0, keepdims=True) / N
    var = ex2 - mean * mean
    hn = (h_ref[...] - mean) / jnp.sqrt(var + 1e-5) * g_ref[...] + b_ref[...]
    o_ref[...] = jax.nn.relu(hn)


def _bn_relu(h, stats, gamma, beta):
    nblk = N // _LIN_BLK
    return pl.pallas_call(
        _bn_kernel,
        grid=(nblk,),
        in_specs=[
            pl.BlockSpec((_LIN_BLK, D), lambda i: (i, 0)),
            pl.BlockSpec(stats.shape, lambda i: (0, 0, 0)),
            pl.BlockSpec((1, D), lambda i: (0, 0)),
            pl.BlockSpec((1, D), lambda i: (0, 0)),
        ],
        out_specs=pl.BlockSpec((_LIN_BLK, D), lambda i: (i, 0)),
        out_shape=jax.ShapeDtypeStruct((N, D), jnp.float32),
        compiler_params=pltpu.CompilerParams(
            dimension_semantics=("parallel",)),
    )(h, stats, gamma, beta)


def kernel(p, u, o, W_pre, b_pre, W_var, b_var, bn_gamma, bn_beta):
    u_pre = _linear(u, W_pre, b_pre.reshape(1, D))

    pb = p.astype(jnp.bfloat16)
    idx = _knn(p, pb.T, p.T)

    u_n = _sc_gather(u_pre, idx.reshape(1, N * NS))
    u_n = u_n.reshape(N, NS, D)

    h, stats = _lap_varphi(u_n, u_pre, W_var, b_var.reshape(1, D))
    u_tt = _bn_relu(h, stats, bn_gamma.reshape(1, D), bn_beta.reshape(1, D))
    return (p, u_tt, o)


# speedup vs baseline: 1.1518x; 1.1518x over previous
import jax
import jax.numpy as jnp
from jax.experimental import pallas as pl
from jax.experimental.pallas import tpu as pltpu
from jax.experimental.pallas import tpu_sc as plsc

N = 8192
D = 128
NS = 16

_LIN_BLK = 512
_KNN_BLK = 256
_GATHER_WIN = 256


def _linear_kernel(x_ref, w_ref, b_ref, o_ref):
    xb = x_ref[...].astype(jnp.bfloat16)
    wb = w_ref[...].astype(jnp.bfloat16)
    acc = jax.lax.dot_general(xb, wb, (((1,), (1,)), ((), ())),
                              preferred_element_type=jnp.float32)
    o_ref[...] = acc + b_ref[...]


def _linear(x, w, b):
    nblk = x.shape[0] // _LIN_BLK
    return pl.pallas_call(
        _linear_kernel,
        grid=(nblk,),
        in_specs=[
            pl.BlockSpec((_LIN_BLK, D), lambda i: (i, 0)),
            pl.BlockSpec((D, D), lambda i: (0, 0)),
            pl.BlockSpec((1, D), lambda i: (0, 0)),
        ],
        out_specs=pl.BlockSpec((_LIN_BLK, D), lambda i: (i, 0)),
        out_shape=jax.ShapeDtypeStruct((x.shape[0], D), jnp.float32),
        compiler_params=pltpu.CompilerParams(
            dimension_semantics=("parallel",)),
    )(x, w, b)


def _knn_kernel(p_ref, pbt_ref, pt_ref, idx_ref):
    B = p_ref.shape[0]
    p_blk = p_ref[...]
    sq_q = jnp.sum(p_blk * p_blk, axis=1, keepdims=True)
    pt = pt_ref[...]
    sq_k = jnp.sum(pt * pt, axis=0, keepdims=True)

    qb = p_blk.astype(jnp.bfloat16).astype(jnp.float32)
    kb = pbt_ref[...].astype(jnp.float32)
    g = qb[:, 0:1] * kb[0:1, :]
    g = g + qb[:, 1:2] * kb[1:2, :]
    g = g + qb[:, 2:3] * kb[2:3, :]
    d2 = (sq_q + sq_k) - 2.0 * g

    iota_f = jax.lax.broadcasted_iota(jnp.int32, (B, N), 1).astype(jnp.float32)
    big_f = jnp.float32(1e9)
    cols = []
    for t in range(NS):
        m = jnp.min(d2, axis=1, keepdims=True)
        cand = jnp.where(d2 <= m, iota_f, big_f)
        mi = jnp.min(cand, axis=1, keepdims=True)
        cols.append(mi)
        if t < NS - 1:
            d2 = jnp.where(cand == mi, jnp.float32(jnp.inf), d2)
    idx_ref[...] = jnp.concatenate(cols, axis=1).astype(jnp.int32)


def _knn(p, pbt, pt):
    nblk = N // _KNN_BLK
    return pl.pallas_call(
        _knn_kernel,
        grid=(nblk,),
        in_specs=[
            pl.BlockSpec((_KNN_BLK, 3), lambda i: (i, 0)),
            pl.BlockSpec((3, N), lambda i: (0, 0)),
            pl.BlockSpec((3, N), lambda i: (0, 0)),
        ],
        out_specs=pl.BlockSpec((_KNN_BLK, NS), lambda i: (i, 0)),
        out_shape=jax.ShapeDtypeStruct((N, NS), jnp.int32),
        compiler_params=pltpu.CompilerParams(
            dimension_semantics=("parallel",)),
    )(p, pbt, pt)


def _sc_gather(table, idx_flat):
    num_indices = idx_flat.shape[1]
    mesh = plsc.VectorSubcoreMesh(core_axis_name="core",
                                  subcore_axis_name="subcore")

    @pl.kernel(
        out_type=jax.ShapeDtypeStruct((num_indices, D), table.dtype),
        mesh=mesh,
    )
    def _gather_kernel(x_hbm, i_hbm, o_hbm):
        def body(i_vmem, o_vmem):
            pltpu.sync_copy(x_hbm.at[i_vmem.at[0]], o_vmem)

        pltpu.emit_pipeline(
            body,
            grid=(num_indices // _GATHER_WIN,),
            in_specs=[pl.BlockSpec((1, _GATHER_WIN), index_map=lambda i: (0, i))],
            out_specs=[pl.BlockSpec((_GATHER_WIN, D), index_map=lambda i: (i, 0))],
            core_axis_name=("core", "subcore"),
            dimension_semantics=(pltpu.PARALLEL,),
        )(i_hbm, o_hbm)

    return _gather_kernel(table, idx_flat)


def _lap_varphi_kernel(un_ref, up_ref, w_ref, b_ref, h_ref, stats_ref):
    un = un_ref[...]
    uq = up_ref[...]
    lap = jnp.mean(jax.nn.relu(un - uq[:, None, :]), axis=1)
    lb = lap.astype(jnp.bfloat16)
    wb = w_ref[...].astype(jnp.bfloat16)
    h = jax.lax.dot_general(lb, wb, (((1,), (1,)), ((), ())),
                            preferred_element_type=jnp.float32)
    h = h + b_ref[...]
    h_ref[...] = h
    stats_ref[0, 0, :] = jnp.sum(h, axis=0)
    stats_ref[0, 1, :] = jnp.sum(h * h, axis=0)


def _lap_varphi(u_n, u_pre, w_var, b_var):
    nblk = N // _LIN_BLK
    return pl.pallas_call(
        _lap_varphi_kernel,
        grid=(nblk,),
        in_specs=[
            pl.BlockSpec((_LIN_BLK, NS, D), lambda i: (i, 0, 0)),
            pl.BlockSpec((_LIN_BLK, D), lambda i: (i, 0)),
            pl.BlockSpec((D, D), lambda i: (0, 0)),
            pl.BlockSpec((1, D), lambda i: (0, 0)),
        ],
        out_specs=[
            pl.BlockSpec((_LIN_BLK, D), lambda i: (i, 0)),
            pl.BlockSpec((1, 2, D), lambda i: (i, 0, 0)),
        ],
        out_shape=[
            jax.ShapeDtypeStruct((N, D), jnp.float32),
            jax.ShapeDtypeStruct((nblk, 2, D), jnp.float32),
        ],
        compiler_params=pltpu.CompilerParams(
            dimension_semantics=("parallel",)),
    )(u_n, u_pre, w_var, b_var)


def _bn_kernel(h_ref, stats_ref, g_ref, b_ref, o_ref):
    stats = stats_ref[...]
    mean = jnp.sum(stats[:, 0, :], axis=0, keepdims=True) / N
    ex2 = jnp.sum(stats[:, 1, :], axis=0, keepdims=True) / N
    var = ex2 - mean * mean
    hn = (h_ref[...] - mean) / jnp.sqrt(var + 1e-5) * g_ref[...] + b_ref[...]
    o_ref[...] = jax.nn.relu(hn)


def _bn_relu(h, stats, gamma, beta):
    nblk = N // _LIN_BLK
    return pl.pallas_call(
        _bn_kernel,
        grid=(nblk,),
        in_specs=[
            pl.BlockSpec((_LIN_BLK, D), lambda i: (i, 0)),
            pl.BlockSpec(stats.shape, lambda i: (0, 0, 0)),
            pl.BlockSpec((1, D), lambda i: (0, 0)),
            pl.BlockSpec((1, D), lambda i: (0, 0)),
        ],
        out_specs=pl.BlockSpec((_LIN_BLK, D), lambda i: (i, 0)),
        out_shape=jax.ShapeDtypeStruct((N, D), jnp.float32),
        compiler_params=pltpu.CompilerParams(
            dimension_semantics=("parallel",)),
    )(h, stats, gamma, beta)


def kernel(p, u, o, W_pre, b_pre, W_var, b_var, bn_gamma, bn_beta):
    u_pre = _linear(u, W_pre, b_pre.reshape(1, D))

    pb = p.astype(jnp.bfloat16)
    idx = _knn(p, pb.T, p.T)

    u_n = _sc_gather(u_pre, idx.reshape(1, N * NS))
    u_n = u_n.reshape(N, NS, D)

    h, stats = _lap_varphi(u_n, u_pre, W_var, b_var.reshape(1, D))
    u_tt = _bn_relu(h, stats, bn_gamma.reshape(1, D), bn_beta.reshape(1, D))
    return (p, u_tt, o)
